# baseline (speedup 1.0000x reference)
import functools

import jax
import jax.numpy as jnp
from jax import lax
from jax.experimental import pallas as pl
from jax.experimental.pallas import tpu as pltpu
from jax.experimental.pallas import tpu_sc as plsc

N_NODES = 10000
N_EDGES = 320000
N_GRAPHS = 64
NC = 2
NS = 16
NW = NC * NS
EPT = N_EDGES // NW
CHUNK = 80
NCHUNK = EPT // CHUNK
NPAIR = NCHUNK // 2
N_PAD = 10240
ROWS_PT = N_PAD // NS
BLK = 2048
NBLK = N_PAD // BLK
EPS = 1e-5


@functools.lru_cache(maxsize=None)
def _make_agg(F):
    @functools.partial(
        pl.kernel,
        out_type=jax.ShapeDtypeStruct((NC * N_PAD, F), jnp.float32),
        mesh=plsc.VectorSubcoreMesh(
            core_axis_name="c", subcore_axis_name="s",
            num_cores=NC, num_subcores=NS),
        scratch_types=[
            pltpu.VMEM((EPT,), jnp.int32),
            pltpu.VMEM((NCHUNK, CHUNK), jnp.int32),
            pltpu.VMEM((CHUNK, F), jnp.float32),
            pltpu.VMEM((CHUNK, F), jnp.float32),
            pltpu.VMEM_SHARED((N_PAD, F), jnp.float32),
            pltpu.SemaphoreType.DMA,
            pltpu.SemaphoreType.DMA,
        ],
    )
    def agg(h_hbm, src_hbm, dst_hbm, zeros_hbm, out_hbm,
            src_v, dst_v, rows0, rows1, acc_sh, sg0, sg1):
        c = lax.axis_index("c")
        s = lax.axis_index("s")
        wid = s * NC + c
        pltpu.sync_copy(zeros_hbm, acc_sh.at[pl.ds(s * ROWS_PT, ROWS_PT)])
        pltpu.sync_copy(src_hbm.at[wid], src_v)
        pltpu.sync_copy(dst_hbm.at[wid], dst_v)
        plsc.subcore_barrier()

        def g_start(j, buf, sem):
            pltpu.async_copy(h_hbm.at[src_v.at[pl.ds(j * CHUNK, CHUNK)]],
                             buf, sem)

        def g_wait(buf, sem):
            pltpu.make_async_copy(h_hbm.at[src_v.at[pl.ds(0, CHUNK)]],
                                  buf, sem).wait()

        g_start(0, rows0, sg0)

        def pair(p, carry):
            a = 2 * p
            g_wait(rows0, sg0)
            g_start(a + 1, rows1, sg1)
            pltpu.sync_copy(rows0, acc_sh.at[dst_v.at[a]], add=True)
            g_wait(rows1, sg1)

            @pl.when(a + 2 < NCHUNK)
            def _():
                g_start(a + 2, rows0, sg0)

            pltpu.sync_copy(rows1, acc_sh.at[dst_v.at[a + 1]], add=True)
            return carry

        lax.fori_loop(0, NPAIR, pair, 0)
        g_wait(rows0, sg0)
        pltpu.sync_copy(rows0, acc_sh.at[dst_v.at[NCHUNK - 1]], add=True)
        plsc.subcore_barrier()
        pltpu.sync_copy(
            acc_sh.at[pl.ds(s * ROWS_PT, ROWS_PT)],
            out_hbm.at[pl.ds(c * N_PAD + s * ROWS_PT, ROWS_PT)])

    return agg


def _elu(z):
    return jnp.where(z > 0, z, jnp.exp(jnp.minimum(z, 0.0)) - 1.0)


def _pre_body(x_ref, wT_ref, b_ref, gs_ref, gb_ref, wrel0T_ref,
              out_ref, y_ref):
    z = jnp.dot(x_ref[...], wT_ref[...], preferred_element_type=jnp.float32)
    z = _elu(z + b_ref[...])
    h0 = z * gs_ref[...] + gb_ref[...]
    out_ref[...] = h0
    y_ref[...] = jnp.dot(h0, wrel0T_ref[...], preferred_element_type=jnp.float32)


def _pre_call(x, wT, b, gs, gb, wrel0T):
    return pl.pallas_call(
        _pre_body,
        grid=(NBLK,),
        in_specs=[
            pl.BlockSpec((BLK, 38), lambda i: (i, 0)),
            pl.BlockSpec((38, 16), lambda i: (0, 0)),
            pl.BlockSpec((1, 16), lambda i: (0, 0)),
            pl.BlockSpec((1, 16), lambda i: (0, 0)),
            pl.BlockSpec((1, 16), lambda i: (0, 0)),
            pl.BlockSpec((16, 128), lambda i: (0, 0)),
        ],
        out_specs=[
            pl.BlockSpec((BLK, 16), lambda i: (i, 0)),
            pl.BlockSpec((BLK, 128), lambda i: (i, 0)),
        ],
        out_shape=[
            jax.ShapeDtypeStruct((N_PAD, 16), jnp.float32),
            jax.ShapeDtypeStruct((N_PAD, 128), jnp.float32),
        ],
    )(x, wT, b, gs, gb, wrel0T)


def _layer_math(agg0, agg1, h, wrelT, brel, wrootT, gs, gb):
    z = jnp.dot(agg0[...] + agg1[...], wrelT[...],
                preferred_element_type=jnp.float32)
    z = z + jnp.dot(h[...], wrootT[...], preferred_element_type=jnp.float32)
    z = _elu(z + brel[...])
    return z * gs[...] + gb[...]


def _layer_body(agg0, agg1, h, wrelT, brel, wrootT, gs, gb, hout):
    hout[...] = _layer_math(agg0, agg1, h, wrelT, brel, wrootT, gs, gb)


def _layer_pool_body(agg0, agg1, h, wrelT, brel, wrootT, gs, gb, bi_ref,
                     hout, pout, acc):
    i = pl.program_id(0)
    z = _layer_math(agg0, agg1, h, wrelT, brel, wrootT, gs, gb)
    hout[...] = z

    @pl.when(i == 0)
    def _():
        acc[...] = jnp.full((N_GRAPHS, 128), -jnp.inf, jnp.float32)

    bi = bi_ref[0]
    lo = bi_ref[0, 0, 0]
    hi = bi_ref[0, BLK - 1, 0]
    gidx = lax.broadcasted_iota(jnp.int32, (N_GRAPHS, 128), 0)

    def gupd(g, carry):
        m = bi == g
        colmax = jnp.max(jnp.where(m, z, -jnp.inf), axis=0)
        acc[...] = jnp.maximum(
            acc[...], jnp.where(gidx == g, colmax[None, :], -jnp.inf))
        return carry

    lax.fori_loop(lo, hi + 1, gupd, 0)

    @pl.when(i == NBLK - 1)
    def _():
        pout[...] = acc[...]


def _layer_specs(F_agg, F_h, with_pool):
    in_specs = [
        pl.BlockSpec((BLK, F_agg), lambda i: (i, 0)),
        pl.BlockSpec((BLK, F_agg), lambda i: (i, 0)),
        pl.BlockSpec((BLK, F_h), lambda i: (i, 0)),
        pl.BlockSpec((F_agg, 128), lambda i: (0, 0)),
        pl.BlockSpec((1, 128), lambda i: (0, 0)),
        pl.BlockSpec((F_h, 128), lambda i: (0, 0)),
        pl.BlockSpec((1, 128), lambda i: (0, 0)),
        pl.BlockSpec((1, 128), lambda i: (0, 0)),
    ]
    if with_pool:
        in_specs.append(pl.BlockSpec((1, BLK, 1), lambda i: (i, 0, 0)))
    return in_specs


def _layer_call(F_h, agg0, agg1, h, wrelT, brel, wrootT, gs, gb):
    return pl.pallas_call(
        _layer_body,
        grid=(NBLK,),
        in_specs=_layer_specs(128, F_h, False),
        out_specs=pl.BlockSpec((BLK, 128), lambda i: (i, 0)),
        out_shape=jax.ShapeDtypeStruct((N_PAD, 128), jnp.float32),
    )(agg0, agg1, h, wrelT, brel, wrootT, gs, gb)


def _layer_pool_call(F_h, agg0, agg1, h, wrelT, brel, wrootT, gs, gb, bi3):
    return pl.pallas_call(
        _layer_pool_body,
        grid=(NBLK,),
        in_specs=_layer_specs(128, F_h, True),
        out_specs=[
            pl.BlockSpec((BLK, 128), lambda i: (i, 0)),
            pl.BlockSpec((N_GRAPHS, 128), lambda i: (0, 0)),
        ],
        out_shape=[
            jax.ShapeDtypeStruct((N_PAD, 128), jnp.float32),
            jax.ShapeDtypeStruct((N_GRAPHS, 128), jnp.float32),
        ],
        scratch_shapes=[pltpu.VMEM((N_GRAPHS, 128), jnp.float32)],
    )(agg0, agg1, h, wrelT, brel, wrootT, gs, gb, bi3)


def _mlp_body(p3, p5, p7, w0, b0, w1, b1, wf, bf, out):
    w0v = w0[...]
    z = jnp.dot(p3[...], w0v[0:128, :], preferred_element_type=jnp.float32)
    z = z + jnp.dot(p5[...], w0v[128:256, :], preferred_element_type=jnp.float32)
    z = z + jnp.dot(p7[...], w0v[256:384, :], preferred_element_type=jnp.float32)
    z = _elu(z + b0[...])
    z = _elu(jnp.dot(z, w1[...], preferred_element_type=jnp.float32) + b1[...])
    z = jnp.dot(z, wf[...], preferred_element_type=jnp.float32) + bf[...]
    m = jnp.max(z, axis=1, keepdims=True)
    lse = jnp.log(jnp.sum(jnp.exp(z - m), axis=1, keepdims=True)) + m
    out[...] = z - lse


def _mlp_call(p3, p5, p7, w0, b0, w1, b1, wf, bf):
    return pl.pallas_call(
        _mlp_body,
        out_shape=jax.ShapeDtypeStruct((N_GRAPHS, 3), jnp.float32),
    )(p3, p5, p7, w0, b0, w1, b1, wf, bf)


def kernel(x, pre_lin_W, pre_lin_b, pre_bn_g, pre_bn_b, conv0_Wrel,
           conv0_brel, conv0_Wroot, convs_Wrel, convs_brel, convs_Wroot,
           bn_g, bn_b, lin0_W, lin0_b, lin1_W, lin1_b, linf_W, linf_b,
           edge_index, batch_index):
    f32 = jnp.float32
    inv = 1.0 / jnp.sqrt(jnp.asarray(1.0 + EPS, f32))

    preWT = jnp.zeros((38, 16), f32).at[:, :3].set(pre_lin_W.T)
    preb = jnp.zeros((1, 16), f32).at[0, :3].set(pre_lin_b)
    pregs = jnp.zeros((1, 16), f32).at[0, :3].set(pre_bn_g * inv)
    pregb = jnp.zeros((1, 16), f32).at[0, :3].set(pre_bn_b)
    wrel0T = jnp.zeros((16, 128), f32).at[:3, :].set(conv0_Wrel.T)
    xp = jnp.zeros((N_PAD, 38), f32).at[:N_NODES].set(x)
    h, y0 = _pre_call(xp, preWT, preb, pregs, pregb, wrel0T)

    src3 = edge_index[0].reshape(NW, EPT)
    dst3 = edge_index[1].reshape(NW, NCHUNK, CHUNK)
    zeros128 = jnp.zeros((ROWS_PT, 128), f32)
    bip = jnp.full((N_PAD,), N_GRAPHS, jnp.int32).at[:N_NODES].set(batch_index)
    bi3 = bip.reshape(NBLK, BLK, 1)

    pooled = {}
    for i in range(7):
        if i == 0:
            agg = _make_agg(128)(y0, src3, dst3, zeros128)
            wrelT = jnp.eye(128, dtype=f32)
            wrootT = jnp.zeros((16, 128), f32).at[:3, :].set(conv0_Wroot.T)
            brel = conv0_brel.reshape(1, 128)
            F_h = 16
        else:
            agg = _make_agg(128)(h, src3, dst3, zeros128)
            wrelT = convs_Wrel[i - 1].T
            wrootT = convs_Wroot[i - 1].T
            brel = convs_brel[i - 1].reshape(1, 128)
            F_h = 128
        gs = (bn_g[i] * inv).reshape(1, 128)
        gb = bn_b[i].reshape(1, 128)
        args = (agg[:N_PAD], agg[N_PAD:], h, wrelT, brel, wrootT, gs, gb)
        if i in (2, 4, 6):
            h, p = _layer_pool_call(F_h, *args, bi3)
            pooled[i] = p
        else:
            h = _layer_call(F_h, *args)

    return _mlp_call(
        pooled[2], pooled[4], pooled[6],
        lin0_W.T, lin0_b.reshape(1, 32),
        lin1_W.T, lin1_b.reshape(1, 8),
        linf_W.T, linf_b.reshape(1, 3))


# speedup vs baseline: 1.0372x; 1.0372x over previous
import functools

import jax
import jax.numpy as jnp
from jax import lax
from jax.experimental import pallas as pl
from jax.experimental.pallas import tpu as pltpu
from jax.experimental.pallas import tpu_sc as plsc

N_NODES = 10000
N_EDGES = 320000
N_GRAPHS = 64
NC = 2
NS = 16
NW = NC * NS
EPT = N_EDGES // NW
CHUNK = 80
NCHUNK = EPT // CHUNK
NPAIR = NCHUNK // 2
N_PAD = 10240
ROWS_PT = N_PAD // NS
BLK = 2048
NBLK = N_PAD // BLK
EPS = 1e-5


@functools.lru_cache(maxsize=None)
def _make_agg(F):
    @functools.partial(
        pl.kernel,
        out_type=jax.ShapeDtypeStruct((NC * N_PAD, F), jnp.float32),
        mesh=plsc.VectorSubcoreMesh(
            core_axis_name="c", subcore_axis_name="s",
            num_cores=NC, num_subcores=NS),
        scratch_types=[
            pltpu.VMEM((EPT,), jnp.int32),
            pltpu.VMEM((NCHUNK, CHUNK), jnp.int32),
            pltpu.VMEM((CHUNK, F), jnp.float32),
            pltpu.VMEM((CHUNK, F), jnp.float32),
            pltpu.VMEM_SHARED((N_PAD, F), jnp.float32),
            pltpu.SemaphoreType.DMA,
            pltpu.SemaphoreType.DMA,
        ],
    )
    def agg(h_hbm, src_hbm, dst_hbm, zeros_hbm, out_hbm,
            src_v, dst_v, rows0, rows1, acc_sh, sg0, sg1):
        c = lax.axis_index("c")
        s = lax.axis_index("s")
        wid = s * NC + c
        pltpu.sync_copy(zeros_hbm, acc_sh.at[pl.ds(s * ROWS_PT, ROWS_PT)])
        pltpu.sync_copy(src_hbm.at[wid], src_v)
        pltpu.sync_copy(dst_hbm.at[wid], dst_v)
        plsc.subcore_barrier()

        def g_start(j, buf, sem):
            pltpu.async_copy(h_hbm.at[src_v.at[pl.ds(j * CHUNK, CHUNK)]],
                             buf, sem)

        def g_wait(buf, sem):
            pltpu.make_async_copy(h_hbm.at[src_v.at[pl.ds(0, CHUNK)]],
                                  buf, sem).wait()

        g_start(0, rows0, sg0)

        def pair(p, carry):
            a = 2 * p
            g_wait(rows0, sg0)
            g_start(a + 1, rows1, sg1)
            pltpu.sync_copy(rows0, acc_sh.at[dst_v.at[a]], add=True)
            g_wait(rows1, sg1)

            @pl.when(a + 2 < NCHUNK)
            def _():
                g_start(a + 2, rows0, sg0)

            pltpu.sync_copy(rows1, acc_sh.at[dst_v.at[a + 1]], add=True)
            return carry

        lax.fori_loop(0, NPAIR, pair, 0)
        g_wait(rows0, sg0)
        pltpu.sync_copy(rows0, acc_sh.at[dst_v.at[NCHUNK - 1]], add=True)
        plsc.subcore_barrier()
        pltpu.sync_copy(
            acc_sh.at[pl.ds(s * ROWS_PT, ROWS_PT)],
            out_hbm.at[pl.ds(c * N_PAD + s * ROWS_PT, ROWS_PT)])

    return agg


def _elu(z):
    return jnp.where(z > 0, z, jnp.exp(jnp.minimum(z, 0.0)) - 1.0)


def _pre_body(x_ref, wT_ref, b_ref, gs_ref, gb_ref, wrel0T_ref,
              out_ref, y_ref):
    z = jnp.dot(x_ref[...], wT_ref[...], preferred_element_type=jnp.float32)
    z = _elu(z + b_ref[...])
    h0 = z * gs_ref[...] + gb_ref[...]
    out_ref[...] = h0
    y_ref[...] = jnp.dot(h0, wrel0T_ref[...], preferred_element_type=jnp.float32)


def _pre_call(x, wT, b, gs, gb, wrel0T):
    return pl.pallas_call(
        _pre_body,
        grid=(NBLK,),
        in_specs=[
            pl.BlockSpec((BLK, 38), lambda i: (i, 0)),
            pl.BlockSpec((38, 16), lambda i: (0, 0)),
            pl.BlockSpec((1, 16), lambda i: (0, 0)),
            pl.BlockSpec((1, 16), lambda i: (0, 0)),
            pl.BlockSpec((1, 16), lambda i: (0, 0)),
            pl.BlockSpec((16, 128), lambda i: (0, 0)),
        ],
        out_specs=[
            pl.BlockSpec((BLK, 16), lambda i: (i, 0)),
            pl.BlockSpec((BLK, 128), lambda i: (i, 0)),
        ],
        out_shape=[
            jax.ShapeDtypeStruct((N_PAD, 16), jnp.float32),
            jax.ShapeDtypeStruct((N_PAD, 128), jnp.float32),
        ],
    )(x, wT, b, gs, gb, wrel0T)


def _layer_math(agg0, agg1, h, wrelT, brel, wrootT, gs, gb):
    z = jnp.dot(agg0[...] + agg1[...], wrelT[...],
                preferred_element_type=jnp.float32)
    z = z + jnp.dot(h[...], wrootT[...], preferred_element_type=jnp.float32)
    z = _elu(z + brel[...])
    return z * gs[...] + gb[...]


def _layer_body(agg0, agg1, h, wrelT, brel, wrootT, gs, gb, hout):
    hout[...] = _layer_math(agg0, agg1, h, wrelT, brel, wrootT, gs, gb)


def _layer_pool_body(agg0, agg1, h, wrelT, brel, wrootT, gs, gb, bi_ref,
                     hout, pout, acc):
    i = pl.program_id(0)
    z = _layer_math(agg0, agg1, h, wrelT, brel, wrootT, gs, gb)
    hout[...] = z

    @pl.when(i == 0)
    def _():
        acc[...] = jnp.full((N_GRAPHS, 128), -jnp.inf, jnp.float32)

    bi = bi_ref[0]
    lo = bi_ref[0, 0, 0]
    hi = bi_ref[0, BLK - 1, 0]
    gidx = lax.broadcasted_iota(jnp.int32, (N_GRAPHS, 128), 0)

    def gupd(g, carry):
        m = bi == g
        colmax = jnp.max(jnp.where(m, z, -jnp.inf), axis=0)
        acc[...] = jnp.maximum(
            acc[...], jnp.where(gidx == g, colmax[None, :], -jnp.inf))
        return carry

    lax.fori_loop(lo, hi + 1, gupd, 0)

    @pl.when(i == NBLK - 1)
    def _():
        pout[...] = acc[...]


def _layer_specs(F_agg, F_h, with_pool):
    in_specs = [
        pl.BlockSpec((BLK, F_agg), lambda i: (i, 0)),
        pl.BlockSpec((BLK, F_agg), lambda i: (i + NBLK, 0)),
        pl.BlockSpec((BLK, F_h), lambda i: (i, 0)),
        pl.BlockSpec((F_agg, 128), lambda i: (0, 0)),
        pl.BlockSpec((1, 128), lambda i: (0, 0)),
        pl.BlockSpec((F_h, 128), lambda i: (0, 0)),
        pl.BlockSpec((1, 128), lambda i: (0, 0)),
        pl.BlockSpec((1, 128), lambda i: (0, 0)),
    ]
    if with_pool:
        in_specs.append(pl.BlockSpec((1, BLK, 1), lambda i: (i, 0, 0)))
    return in_specs


def _layer_call(F_h, agg0, agg1, h, wrelT, brel, wrootT, gs, gb):
    return pl.pallas_call(
        _layer_body,
        grid=(NBLK,),
        in_specs=_layer_specs(128, F_h, False),
        out_specs=pl.BlockSpec((BLK, 128), lambda i: (i, 0)),
        out_shape=jax.ShapeDtypeStruct((N_PAD, 128), jnp.float32),
    )(agg0, agg1, h, wrelT, brel, wrootT, gs, gb)


def _layer_pool_call(F_h, agg0, agg1, h, wrelT, brel, wrootT, gs, gb, bi3):
    return pl.pallas_call(
        _layer_pool_body,
        grid=(NBLK,),
        in_specs=_layer_specs(128, F_h, True),
        out_specs=[
            pl.BlockSpec((BLK, 128), lambda i: (i, 0)),
            pl.BlockSpec((N_GRAPHS, 128), lambda i: (0, 0)),
        ],
        out_shape=[
            jax.ShapeDtypeStruct((N_PAD, 128), jnp.float32),
            jax.ShapeDtypeStruct((N_GRAPHS, 128), jnp.float32),
        ],
        scratch_shapes=[pltpu.VMEM((N_GRAPHS, 128), jnp.float32)],
    )(agg0, agg1, h, wrelT, brel, wrootT, gs, gb, bi3)


def _mlp_body(p3, p5, p7, w0, b0, w1, b1, wf, bf, out):
    w0v = w0[...]
    z = jnp.dot(p3[...], w0v[0:128, :], preferred_element_type=jnp.float32)
    z = z + jnp.dot(p5[...], w0v[128:256, :], preferred_element_type=jnp.float32)
    z = z + jnp.dot(p7[...], w0v[256:384, :], preferred_element_type=jnp.float32)
    z = _elu(z + b0[...])
    z = _elu(jnp.dot(z, w1[...], preferred_element_type=jnp.float32) + b1[...])
    z = jnp.dot(z, wf[...], preferred_element_type=jnp.float32) + bf[...]
    m = jnp.max(z, axis=1, keepdims=True)
    lse = jnp.log(jnp.sum(jnp.exp(z - m), axis=1, keepdims=True)) + m
    out[...] = z - lse


def _mlp_call(p3, p5, p7, w0, b0, w1, b1, wf, bf):
    return pl.pallas_call(
        _mlp_body,
        out_shape=jax.ShapeDtypeStruct((N_GRAPHS, 3), jnp.float32),
    )(p3, p5, p7, w0, b0, w1, b1, wf, bf)


def kernel(x, pre_lin_W, pre_lin_b, pre_bn_g, pre_bn_b, conv0_Wrel,
           conv0_brel, conv0_Wroot, convs_Wrel, convs_brel, convs_Wroot,
           bn_g, bn_b, lin0_W, lin0_b, lin1_W, lin1_b, linf_W, linf_b,
           edge_index, batch_index):
    f32 = jnp.float32
    inv = 1.0 / jnp.sqrt(jnp.asarray(1.0 + EPS, f32))

    preWT = jnp.zeros((38, 16), f32).at[:, :3].set(pre_lin_W.T)
    preb = jnp.zeros((1, 16), f32).at[0, :3].set(pre_lin_b)
    pregs = jnp.zeros((1, 16), f32).at[0, :3].set(pre_bn_g * inv)
    pregb = jnp.zeros((1, 16), f32).at[0, :3].set(pre_bn_b)
    wrel0T = jnp.zeros((16, 128), f32).at[:3, :].set(conv0_Wrel.T)
    xp = jnp.zeros((N_PAD, 38), f32).at[:N_NODES].set(x)
    h, y0 = _pre_call(xp, preWT, preb, pregs, pregb, wrel0T)

    src3 = edge_index[0].reshape(NW, EPT)
    dst3 = edge_index[1].reshape(NW, NCHUNK, CHUNK)
    zeros128 = jnp.zeros((ROWS_PT, 128), f32)
    bip = jnp.full((N_PAD,), N_GRAPHS, jnp.int32).at[:N_NODES].set(batch_index)
    bi3 = bip.reshape(NBLK, BLK, 1)

    pooled = {}
    for i in range(7):
        if i == 0:
            agg = _make_agg(128)(y0, src3, dst3, zeros128)
            wrelT = jnp.eye(128, dtype=f32)
            wrootT = jnp.zeros((16, 128), f32).at[:3, :].set(conv0_Wroot.T)
            brel = conv0_brel.reshape(1, 128)
            F_h = 16
        else:
            agg = _make_agg(128)(h, src3, dst3, zeros128)
            wrelT = convs_Wrel[i - 1].T
            wrootT = convs_Wroot[i - 1].T
            brel = convs_brel[i - 1].reshape(1, 128)
            F_h = 128
        gs = (bn_g[i] * inv).reshape(1, 128)
        gb = bn_b[i].reshape(1, 128)
        args = (agg, agg, h, wrelT, brel, wrootT, gs, gb)
        if i in (2, 4, 6):
            h, p = _layer_pool_call(F_h, *args, bi3)
            pooled[i] = p
        else:
            h = _layer_call(F_h, *args)

    return _mlp_call(
        pooled[2], pooled[4], pooled[6],
        lin0_W.T, lin0_b.reshape(1, 32),
        lin1_W.T, lin1_b.reshape(1, 8),
        linf_W.T, linf_b.reshape(1, 3))


# speedup vs baseline: 1.0976x; 1.0582x over previous
import functools

import jax
import jax.numpy as jnp
from jax import lax
from jax.experimental import pallas as pl
from jax.experimental.pallas import tpu as pltpu
from jax.experimental.pallas import tpu_sc as plsc

N_NODES = 10000
N_EDGES = 320000
N_GRAPHS = 64
NC = 2
NS = 16
NW = NC * NS
CHUNK = 96
EPT = 10176
NCHUNK = EPT // CHUNK
NPAIR = NCHUNK // 2
N_PAD = 10240
ROWS_PT = N_PAD // NS
BLK = 2048
NBLK = N_PAD // BLK
EPS = 1e-5


@functools.lru_cache(maxsize=None)
def _make_agg(F):
    @functools.partial(
        pl.kernel,
        out_type=jax.ShapeDtypeStruct((NC * N_PAD, F), jnp.float32),
        mesh=plsc.VectorSubcoreMesh(
            core_axis_name="c", subcore_axis_name="s",
            num_cores=NC, num_subcores=NS),
        scratch_types=[
            pltpu.VMEM((EPT,), jnp.int32),
            pltpu.VMEM((NCHUNK, CHUNK), jnp.int32),
            pltpu.VMEM((CHUNK, F), jnp.float32),
            pltpu.VMEM((CHUNK, F), jnp.float32),
            pltpu.VMEM_SHARED((N_PAD, F), jnp.float32),
            pltpu.SemaphoreType.DMA,
            pltpu.SemaphoreType.DMA,
        ],
    )
    def agg(h_hbm, src_hbm, dst_hbm, zeros_hbm, out_hbm,
            src_v, dst_v, rows0, rows1, acc_sh, sg0, sg1):
        c = lax.axis_index("c")
        s = lax.axis_index("s")
        wid = s * NC + c
        pltpu.sync_copy(zeros_hbm, acc_sh.at[pl.ds(s * ROWS_PT, ROWS_PT)])
        pltpu.sync_copy(src_hbm.at[wid], src_v)
        pltpu.sync_copy(dst_hbm.at[wid], dst_v)
        plsc.subcore_barrier()

        def g_start(j, buf, sem):
            pltpu.async_copy(h_hbm.at[src_v.at[pl.ds(j * CHUNK, CHUNK)]],
                             buf, sem)

        def g_wait(buf, sem):
            pltpu.make_async_copy(h_hbm.at[src_v.at[pl.ds(0, CHUNK)]],
                                  buf, sem).wait()

        g_start(0, rows0, sg0)

        def pair(p, carry):
            a = 2 * p
            g_wait(rows0, sg0)
            g_start(a + 1, rows1, sg1)
            pltpu.sync_copy(rows0, acc_sh.at[dst_v.at[a]], add=True)
            g_wait(rows1, sg1)

            @pl.when(a + 2 < NCHUNK)
            def _():
                g_start(a + 2, rows0, sg0)

            pltpu.sync_copy(rows1, acc_sh.at[dst_v.at[a + 1]], add=True)
            return carry

        lax.fori_loop(0, NPAIR, pair, 0)
        plsc.subcore_barrier()
        pltpu.sync_copy(
            acc_sh.at[pl.ds(s * ROWS_PT, ROWS_PT)],
            out_hbm.at[pl.ds(c * N_PAD + s * ROWS_PT, ROWS_PT)])

    return agg


def _elu(z):
    return jnp.where(z > 0, z, jnp.exp(jnp.minimum(z, 0.0)) - 1.0)


def _pre_body(x_ref, wT_ref, b_ref, gs_ref, gb_ref, wrel0T_ref,
              out_ref, y_ref):
    z = jnp.dot(x_ref[...], wT_ref[...], preferred_element_type=jnp.float32)
    z = _elu(z + b_ref[...])
    h0 = z * gs_ref[...] + gb_ref[...]
    out_ref[...] = h0
    y_ref[...] = jnp.dot(h0, wrel0T_ref[...], preferred_element_type=jnp.float32)


def _pre_call(x, wT, b, gs, gb, wrel0T):
    return pl.pallas_call(
        _pre_body,
        grid=(NBLK,),
        in_specs=[
            pl.BlockSpec((BLK, 38), lambda i: (i, 0)),
            pl.BlockSpec((38, 16), lambda i: (0, 0)),
            pl.BlockSpec((1, 16), lambda i: (0, 0)),
            pl.BlockSpec((1, 16), lambda i: (0, 0)),
            pl.BlockSpec((1, 16), lambda i: (0, 0)),
            pl.BlockSpec((16, 128), lambda i: (0, 0)),
        ],
        out_specs=[
            pl.BlockSpec((BLK, 16), lambda i: (i, 0)),
            pl.BlockSpec((BLK, 128), lambda i: (i, 0)),
        ],
        out_shape=[
            jax.ShapeDtypeStruct((N_PAD, 16), jnp.float32),
            jax.ShapeDtypeStruct((N_PAD, 128), jnp.float32),
        ],
    )(x, wT, b, gs, gb, wrel0T)


def _layer_math(agg0, agg1, h, wrelT, brel, wrootT, gs, gb):
    z = jnp.dot(agg0[...] + agg1[...], wrelT[...],
                preferred_element_type=jnp.float32)
    z = z + jnp.dot(h[...], wrootT[...], preferred_element_type=jnp.float32)
    z = _elu(z + brel[...])
    return z * gs[...] + gb[...]


def _layer_body(agg0, agg1, h, wrelT, brel, wrootT, gs, gb, hout):
    hout[...] = _layer_math(agg0, agg1, h, wrelT, brel, wrootT, gs, gb)


def _layer_pool_body(agg0, agg1, h, wrelT, brel, wrootT, gs, gb, bi_ref,
                     hout, pout, acc):
    i = pl.program_id(0)
    z = _layer_math(agg0, agg1, h, wrelT, brel, wrootT, gs, gb)
    hout[...] = z

    @pl.when(i == 0)
    def _():
        acc[...] = jnp.full((N_GRAPHS, 128), -jnp.inf, jnp.float32)

    bi = bi_ref[0]
    lo = bi_ref[0, 0, 0]
    hi = bi_ref[0, BLK - 1, 0]
    gidx = lax.broadcasted_iota(jnp.int32, (N_GRAPHS, 128), 0)

    def gupd(g, carry):
        m = bi == g
        colmax = jnp.max(jnp.where(m, z, -jnp.inf), axis=0)
        acc[...] = jnp.maximum(
            acc[...], jnp.where(gidx == g, colmax[None, :], -jnp.inf))
        return carry

    lax.fori_loop(lo, hi + 1, gupd, 0)

    @pl.when(i == NBLK - 1)
    def _():
        pout[...] = acc[...]


def _layer_specs(F_agg, F_h, with_pool):
    in_specs = [
        pl.BlockSpec((BLK, F_agg), lambda i: (i, 0)),
        pl.BlockSpec((BLK, F_agg), lambda i: (i + NBLK, 0)),
        pl.BlockSpec((BLK, F_h), lambda i: (i, 0)),
        pl.BlockSpec((F_agg, 128), lambda i: (0, 0)),
        pl.BlockSpec((1, 128), lambda i: (0, 0)),
        pl.BlockSpec((F_h, 128), lambda i: (0, 0)),
        pl.BlockSpec((1, 128), lambda i: (0, 0)),
        pl.BlockSpec((1, 128), lambda i: (0, 0)),
    ]
    if with_pool:
        in_specs.append(pl.BlockSpec((1, BLK, 1), lambda i: (i, 0, 0)))
    return in_specs


def _layer_call(F_h, agg0, agg1, h, wrelT, brel, wrootT, gs, gb):
    return pl.pallas_call(
        _layer_body,
        grid=(NBLK,),
        in_specs=_layer_specs(128, F_h, False),
        out_specs=pl.BlockSpec((BLK, 128), lambda i: (i, 0)),
        out_shape=jax.ShapeDtypeStruct((N_PAD, 128), jnp.float32),
    )(agg0, agg1, h, wrelT, brel, wrootT, gs, gb)


def _layer_pool_call(F_h, agg0, agg1, h, wrelT, brel, wrootT, gs, gb, bi3):
    return pl.pallas_call(
        _layer_pool_body,
        grid=(NBLK,),
        in_specs=_layer_specs(128, F_h, True),
        out_specs=[
            pl.BlockSpec((BLK, 128), lambda i: (i, 0)),
            pl.BlockSpec((N_GRAPHS, 128), lambda i: (0, 0)),
        ],
        out_shape=[
            jax.ShapeDtypeStruct((N_PAD, 128), jnp.float32),
            jax.ShapeDtypeStruct((N_GRAPHS, 128), jnp.float32),
        ],
        scratch_shapes=[pltpu.VMEM((N_GRAPHS, 128), jnp.float32)],
    )(agg0, agg1, h, wrelT, brel, wrootT, gs, gb, bi3)


def _mlp_body(p3, p5, p7, w0, b0, w1, b1, wf, bf, out):
    w0v = w0[...]
    z = jnp.dot(p3[...], w0v[0:128, :], preferred_element_type=jnp.float32)
    z = z + jnp.dot(p5[...], w0v[128:256, :], preferred_element_type=jnp.float32)
    z = z + jnp.dot(p7[...], w0v[256:384, :], preferred_element_type=jnp.float32)
    z = _elu(z + b0[...])
    z = _elu(jnp.dot(z, w1[...], preferred_element_type=jnp.float32) + b1[...])
    z = jnp.dot(z, wf[...], preferred_element_type=jnp.float32) + bf[...]
    m = jnp.max(z, axis=1, keepdims=True)
    lse = jnp.log(jnp.sum(jnp.exp(z - m), axis=1, keepdims=True)) + m
    out[...] = z - lse


def _mlp_call(p3, p5, p7, w0, b0, w1, b1, wf, bf):
    return pl.pallas_call(
        _mlp_body,
        out_shape=jax.ShapeDtypeStruct((N_GRAPHS, 3), jnp.float32),
    )(p3, p5, p7, w0, b0, w1, b1, wf, bf)


def kernel(x, pre_lin_W, pre_lin_b, pre_bn_g, pre_bn_b, conv0_Wrel,
           conv0_brel, conv0_Wroot, convs_Wrel, convs_brel, convs_Wroot,
           bn_g, bn_b, lin0_W, lin0_b, lin1_W, lin1_b, linf_W, linf_b,
           edge_index, batch_index):
    f32 = jnp.float32
    inv = 1.0 / jnp.sqrt(jnp.asarray(1.0 + EPS, f32))

    preWT = jnp.zeros((38, 16), f32).at[:, :3].set(pre_lin_W.T)
    preb = jnp.zeros((1, 16), f32).at[0, :3].set(pre_lin_b)
    pregs = jnp.zeros((1, 16), f32).at[0, :3].set(pre_bn_g * inv)
    pregb = jnp.zeros((1, 16), f32).at[0, :3].set(pre_bn_b)
    wrel0T = jnp.zeros((16, 128), f32).at[:3, :].set(conv0_Wrel.T)
    xp = jnp.zeros((N_PAD, 38), f32).at[:N_NODES].set(x)
    h, y0 = _pre_call(xp, preWT, preb, pregs, pregb, wrel0T)

    pad_n = NW * EPT - N_EDGES
    padidx = (jnp.arange(pad_n, dtype=jnp.int32) % (N_PAD - N_NODES)) + N_NODES
    src3 = jnp.concatenate([edge_index[0], padidx]).reshape(NW, EPT)
    dst3 = jnp.concatenate([edge_index[1], padidx]).reshape(NW, NCHUNK, CHUNK)
    zeros128 = jnp.zeros((ROWS_PT, 128), f32)
    bip = jnp.full((N_PAD,), N_GRAPHS, jnp.int32).at[:N_NODES].set(batch_index)
    bi3 = bip.reshape(NBLK, BLK, 1)

    pooled = {}
    for i in range(7):
        if i == 0:
            agg = _make_agg(128)(y0, src3, dst3, zeros128)
            wrelT = jnp.eye(128, dtype=f32)
            wrootT = jnp.zeros((16, 128), f32).at[:3, :].set(conv0_Wroot.T)
            brel = conv0_brel.reshape(1, 128)
            F_h = 16
        else:
            agg = _make_agg(128)(h, src3, dst3, zeros128)
            wrelT = convs_Wrel[i - 1].T
            wrootT = convs_Wroot[i - 1].T
            brel = convs_brel[i - 1].reshape(1, 128)
            F_h = 128
        gs = (bn_g[i] * inv).reshape(1, 128)
        gb = bn_b[i].reshape(1, 128)
        args = (agg, agg, h, wrelT, brel, wrootT, gs, gb)
        if i in (2, 4, 6):
            h, p = _layer_pool_call(F_h, *args, bi3)
            pooled[i] = p
        else:
            h = _layer_call(F_h, *args)

    return _mlp_call(
        pooled[2], pooled[4], pooled[6],
        lin0_W.T, lin0_b.reshape(1, 32),
        lin1_W.T, lin1_b.reshape(1, 8),
        linf_W.T, linf_b.reshape(1, 3))


# speedup vs baseline: 1.1925x; 1.0865x over previous
import functools

import jax
import jax.numpy as jnp
from jax import lax
from jax.experimental import pallas as pl
from jax.experimental.pallas import tpu as pltpu
from jax.experimental.pallas import tpu_sc as plsc

N_NODES = 10000
N_EDGES = 320000
N_GRAPHS = 64
NC = 2
NS = 16
NW = NC * NS
CHUNK = 128
EPT = 10240
NCHUNK = EPT // CHUNK
NHALF = NCHUNK // 2
NPAIR_H = NHALF // 2
N_PAD = 10240
ROWS_PT = N_PAD // NS
BLK = 2048
NBLK = N_PAD // BLK
EPS = 1e-5


@functools.lru_cache(maxsize=None)
def _make_agg(F):
    @functools.partial(
        pl.kernel,
        out_type=jax.ShapeDtypeStruct((NC * N_PAD, F), jnp.float32),
        mesh=plsc.VectorSubcoreMesh(
            core_axis_name="c", subcore_axis_name="s",
            num_cores=NC, num_subcores=NS),
        scratch_types=[
            pltpu.VMEM((NHALF * CHUNK,), jnp.int32),
            pltpu.VMEM((NCHUNK, CHUNK), jnp.int32),
            pltpu.VMEM((CHUNK, F), jnp.float32),
            pltpu.VMEM((CHUNK, F), jnp.float32),
            pltpu.VMEM_SHARED((N_PAD, F), jnp.float32),
            pltpu.SemaphoreType.DMA,
            pltpu.SemaphoreType.DMA,
        ],
    )
    def agg(h_hbm, src_hbm, dst_hbm, zeros_hbm, out_hbm,
            src_v, dst_v, rows0, rows1, acc_sh, sg0, sg1):
        c = lax.axis_index("c")
        s = lax.axis_index("s")
        wid = s * NC + c
        pltpu.sync_copy(zeros_hbm, acc_sh.at[pl.ds(s * ROWS_PT, ROWS_PT)])
        pltpu.sync_copy(dst_hbm.at[wid], dst_v)
        plsc.subcore_barrier()

        def g_start(jl, buf, sem):
            pltpu.async_copy(h_hbm.at[src_v.at[pl.ds(jl * CHUNK, CHUNK)]],
                             buf, sem)

        def g_wait(buf, sem):
            pltpu.make_async_copy(h_hbm.at[src_v.at[pl.ds(0, CHUNK)]],
                                  buf, sem).wait()

        for hh in range(2):
            pltpu.sync_copy(
                src_hbm.at[wid, pl.ds(hh * NHALF * CHUNK, NHALF * CHUNK)],
                src_v)
            g_start(0, rows0, sg0)

            def pair(p, carry, hh=hh):
                a = 2 * p
                g_wait(rows0, sg0)
                g_start(a + 1, rows1, sg1)
                pltpu.sync_copy(
                    rows0, acc_sh.at[dst_v.at[hh * NHALF + a]], add=True)
                g_wait(rows1, sg1)

                @pl.when(a + 2 < NHALF)
                def _():
                    g_start(a + 2, rows0, sg0)

                pltpu.sync_copy(
                    rows1, acc_sh.at[dst_v.at[hh * NHALF + a + 1]], add=True)
                return carry

            lax.fori_loop(0, NPAIR_H, pair, 0)
        plsc.subcore_barrier()
        pltpu.sync_copy(
            acc_sh.at[pl.ds(s * ROWS_PT, ROWS_PT)],
            out_hbm.at[pl.ds(c * N_PAD + s * ROWS_PT, ROWS_PT)])

    return agg


def _elu(z):
    return jnp.where(z > 0, z, jnp.exp(jnp.minimum(z, 0.0)) - 1.0)


def _pre_body(x_ref, wT_ref, b_ref, gs_ref, gb_ref, wrel0T_ref,
              out_ref, y_ref):
    z = jnp.dot(x_ref[...], wT_ref[...], preferred_element_type=jnp.float32)
    z = _elu(z + b_ref[...])
    h0 = z * gs_ref[...] + gb_ref[...]
    out_ref[...] = h0
    y_ref[...] = jnp.dot(h0, wrel0T_ref[...], preferred_element_type=jnp.float32)


def _pre_call(x, wT, b, gs, gb, wrel0T):
    return pl.pallas_call(
        _pre_body,
        grid=(NBLK,),
        in_specs=[
            pl.BlockSpec((BLK, 38), lambda i: (i, 0)),
            pl.BlockSpec((38, 16), lambda i: (0, 0)),
            pl.BlockSpec((1, 16), lambda i: (0, 0)),
            pl.BlockSpec((1, 16), lambda i: (0, 0)),
            pl.BlockSpec((1, 16), lambda i: (0, 0)),
            pl.BlockSpec((16, 128), lambda i: (0, 0)),
        ],
        out_specs=[
            pl.BlockSpec((BLK, 16), lambda i: (i, 0)),
            pl.BlockSpec((BLK, 128), lambda i: (i, 0)),
        ],
        out_shape=[
            jax.ShapeDtypeStruct((N_PAD, 16), jnp.float32),
            jax.ShapeDtypeStruct((N_PAD, 128), jnp.float32),
        ],
    )(x, wT, b, gs, gb, wrel0T)


def _layer_math(agg0, agg1, h, wrelT, brel, wrootT, gs, gb):
    z = jnp.dot(agg0[...] + agg1[...], wrelT[...],
                preferred_element_type=jnp.float32)
    z = z + jnp.dot(h[...], wrootT[...], preferred_element_type=jnp.float32)
    z = _elu(z + brel[...])
    return z * gs[...] + gb[...]


def _layer_body(agg0, agg1, h, wrelT, brel, wrootT, gs, gb, hout):
    hout[...] = _layer_math(agg0, agg1, h, wrelT, brel, wrootT, gs, gb)


def _layer_pool_body(agg0, agg1, h, wrelT, brel, wrootT, gs, gb, bi_ref,
                     hout, pout, acc):
    i = pl.program_id(0)
    z = _layer_math(agg0, agg1, h, wrelT, brel, wrootT, gs, gb)
    hout[...] = z

    @pl.when(i == 0)
    def _():
        acc[...] = jnp.full((N_GRAPHS, 128), -jnp.inf, jnp.float32)

    bi = bi_ref[0]
    lo = bi_ref[0, 0, 0]
    hi = bi_ref[0, BLK - 1, 0]
    gidx = lax.broadcasted_iota(jnp.int32, (N_GRAPHS, 128), 0)

    def gupd(g, carry):
        m = bi == g
        colmax = jnp.max(jnp.where(m, z, -jnp.inf), axis=0)
        acc[...] = jnp.maximum(
            acc[...], jnp.where(gidx == g, colmax[None, :], -jnp.inf))
        return carry

    lax.fori_loop(lo, hi + 1, gupd, 0)

    @pl.when(i == NBLK - 1)
    def _():
        pout[...] = acc[...]


def _layer_specs(F_agg, F_h, with_pool):
    in_specs = [
        pl.BlockSpec((BLK, F_agg), lambda i: (i, 0)),
        pl.BlockSpec((BLK, F_agg), lambda i: (i + NBLK, 0)),
        pl.BlockSpec((BLK, F_h), lambda i: (i, 0)),
        pl.BlockSpec((F_agg, 128), lambda i: (0, 0)),
        pl.BlockSpec((1, 128), lambda i: (0, 0)),
        pl.BlockSpec((F_h, 128), lambda i: (0, 0)),
        pl.BlockSpec((1, 128), lambda i: (0, 0)),
        pl.BlockSpec((1, 128), lambda i: (0, 0)),
    ]
    if with_pool:
        in_specs.append(pl.BlockSpec((1, BLK, 1), lambda i: (i, 0, 0)))
    return in_specs


def _layer_call(F_h, agg0, agg1, h, wrelT, brel, wrootT, gs, gb):
    return pl.pallas_call(
        _layer_body,
        grid=(NBLK,),
        in_specs=_layer_specs(128, F_h, False),
        out_specs=pl.BlockSpec((BLK, 128), lambda i: (i, 0)),
        out_shape=jax.ShapeDtypeStruct((N_PAD, 128), jnp.float32),
    )(agg0, agg1, h, wrelT, brel, wrootT, gs, gb)


def _layer_pool_call(F_h, agg0, agg1, h, wrelT, brel, wrootT, gs, gb, bi3):
    return pl.pallas_call(
        _layer_pool_body,
        grid=(NBLK,),
        in_specs=_layer_specs(128, F_h, True),
        out_specs=[
            pl.BlockSpec((BLK, 128), lambda i: (i, 0)),
            pl.BlockSpec((N_GRAPHS, 128), lambda i: (0, 0)),
        ],
        out_shape=[
            jax.ShapeDtypeStruct((N_PAD, 128), jnp.float32),
            jax.ShapeDtypeStruct((N_GRAPHS, 128), jnp.float32),
        ],
        scratch_shapes=[pltpu.VMEM((N_GRAPHS, 128), jnp.float32)],
    )(agg0, agg1, h, wrelT, brel, wrootT, gs, gb, bi3)


def _mlp_body(p3, p5, p7, w0, b0, w1, b1, wf, bf, out):
    w0v = w0[...]
    z = jnp.dot(p3[...], w0v[0:128, :], preferred_element_type=jnp.float32)
    z = z + jnp.dot(p5[...], w0v[128:256, :], preferred_element_type=jnp.float32)
    z = z + jnp.dot(p7[...], w0v[256:384, :], preferred_element_type=jnp.float32)
    z = _elu(z + b0[...])
    z = _elu(jnp.dot(z, w1[...], preferred_element_type=jnp.float32) + b1[...])
    z = jnp.dot(z, wf[...], preferred_element_type=jnp.float32) + bf[...]
    m = jnp.max(z, axis=1, keepdims=True)
    lse = jnp.log(jnp.sum(jnp.exp(z - m), axis=1, keepdims=True)) + m
    out[...] = z - lse


def _mlp_call(p3, p5, p7, w0, b0, w1, b1, wf, bf):
    return pl.pallas_call(
        _mlp_body,
        out_shape=jax.ShapeDtypeStruct((N_GRAPHS, 3), jnp.float32),
    )(p3, p5, p7, w0, b0, w1, b1, wf, bf)


def kernel(x, pre_lin_W, pre_lin_b, pre_bn_g, pre_bn_b, conv0_Wrel,
           conv0_brel, conv0_Wroot, convs_Wrel, convs_brel, convs_Wroot,
           bn_g, bn_b, lin0_W, lin0_b, lin1_W, lin1_b, linf_W, linf_b,
           edge_index, batch_index):
    f32 = jnp.float32
    inv = 1.0 / jnp.sqrt(jnp.asarray(1.0 + EPS, f32))

    preWT = jnp.zeros((38, 16), f32).at[:, :3].set(pre_lin_W.T)
    preb = jnp.zeros((1, 16), f32).at[0, :3].set(pre_lin_b)
    pregs = jnp.zeros((1, 16), f32).at[0, :3].set(pre_bn_g * inv)
    pregb = jnp.zeros((1, 16), f32).at[0, :3].set(pre_bn_b)
    wrel0T = jnp.zeros((16, 128), f32).at[:3, :].set(conv0_Wrel.T)
    xp = jnp.zeros((N_PAD, 38), f32).at[:N_NODES].set(x)
    h, y0 = _pre_call(xp, preWT, preb, pregs, pregb, wrel0T)

    pad_n = NW * EPT - N_EDGES
    padidx = (jnp.arange(pad_n, dtype=jnp.int32) % (N_PAD - N_NODES)) + N_NODES
    src3 = jnp.concatenate([edge_index[0], padidx]).reshape(NW, EPT)
    dst3 = jnp.concatenate([edge_index[1], padidx]).reshape(NW, NCHUNK, CHUNK)
    zeros128 = jnp.zeros((ROWS_PT, 128), f32)
    bip = jnp.full((N_PAD,), N_GRAPHS, jnp.int32).at[:N_NODES].set(batch_index)
    bi3 = bip.reshape(NBLK, BLK, 1)

    pooled = {}
    for i in range(7):
        if i == 0:
            agg = _make_agg(128)(y0, src3, dst3, zeros128)
            wrelT = jnp.eye(128, dtype=f32)
            wrootT = jnp.zeros((16, 128), f32).at[:3, :].set(conv0_Wroot.T)
            brel = conv0_brel.reshape(1, 128)
            F_h = 16
        else:
            agg = _make_agg(128)(h, src3, dst3, zeros128)
            wrelT = convs_Wrel[i - 1].T
            wrootT = convs_Wroot[i - 1].T
            brel = convs_brel[i - 1].reshape(1, 128)
            F_h = 128
        gs = (bn_g[i] * inv).reshape(1, 128)
        gb = bn_b[i].reshape(1, 128)
        args = (agg, agg, h, wrelT, brel, wrootT, gs, gb)
        if i in (2, 4, 6):
            h, p = _layer_pool_call(F_h, *args, bi3)
            pooled[i] = p
        else:
            h = _layer_call(F_h, *args)

    return _mlp_call(
        pooled[2], pooled[4], pooled[6],
        lin0_W.T, lin0_b.reshape(1, 32),
        lin1_W.T, lin1_b.reshape(1, 8),
        linf_W.T, linf_b.reshape(1, 3))
